# baseline (device time: 193288 ns/iter reference)
import jax
import jax.numpy as jnp
from jax import lax
from jax.experimental import pallas as pl
from jax.experimental.pallas import tpu as pltpu

N_DEV = 8
M_CHUNK = 1024
Q = M_CHUNK // 4
N_FLOWS = 4
COMM_DTYPE = jnp.bfloat16


def kernel(x, W1, W2):
    M, K = x.shape
    _, D = W1.shape
    _, F = W2.shape
    n_steps = N_DEV - 1

    def body(x_hbm, w1_ref, w2_ref, out_ref,
             xblk, part, sbuf, comm, w1b, w2b, *sems):
        load_sems = sems[0:2]
        send_sems = sems[2:2 + N_FLOWS]
        recv_sems = sems[2 + N_FLOWS:2 + 2 * N_FLOWS]
        cred_sems = sems[2 + 2 * N_FLOWS:2 + 3 * N_FLOWS]
        my = lax.axis_index("i")
        left = lax.rem(my + N_DEV - 1, N_DEV)
        right = lax.rem(my + 1, N_DEV)

        w1b[...] = w1_ref[...].astype(jnp.bfloat16)
        w2b[...] = w2_ref[...].astype(jnp.bfloat16)

        barrier = pltpu.get_barrier_semaphore()
        for nbr in (left, right):
            pl.semaphore_signal(barrier, inc=1, device_id=(nbr,),
                                device_id_type=pl.DeviceIdType.MESH)
        pl.semaphore_wait(barrier, 2)

        class Obj:
            pass

        dirs = []
        for di, is_r in enumerate((True, False)):
            d = Obj()
            d.di = di
            d.base = 0 if is_r else 512
            if is_r:
                d.rs_chunk = lambda s, my=my: lax.rem(my + N_DEV - s - 1, N_DEV)
            else:
                d.rs_chunk = lambda s, my=my: lax.rem(my + s + 1, N_DEV)
            d.load_cp = {}
            dirs.append(d)

        flow_defs = [(0, 0), (1, 0), (0, 1), (1, 1)]
        flows = []
        for fi, (di, sub) in enumerate(flow_defs):
            f = Obj()
            f.fi = fi
            f.dir = dirs[di]
            f.sub = sub
            is_r = di == 0
            f.qoff = f.dir.base + sub * Q
            f.dst = right if is_r else left
            f.src = left if is_r else right
            f.rs_chunk = f.dir.rs_chunk
            if is_r:
                f.ag_chunk = lambda t, my=my: lax.rem(my + N_DEV - t, N_DEV)
                f.own = lax.rem(my + 1, N_DEV)
            else:
                f.ag_chunk = lambda t, my=my: lax.rem(my + t, N_DEV)
                f.own = lax.rem(my + N_DEV - 1, N_DEV)
            f.rdma = {}
            flows.append(f)

        def start_load(d, li, c):
            cp = pltpu.make_async_copy(
                x_hbm.at[pl.ds(c * M_CHUNK + d.base, 2 * Q), :],
                xblk.at[d.di, li % 2], load_sems[d.di].at[li % 2])
            cp.start()
            d.load_cp[li] = cp

        def gemm1(d, li, next_c):
            d.load_cp[li].wait()
            if next_c is not None:
                start_load(d, li + 1, next_c)
            part[d.di] = jnp.dot(xblk[d.di, li % 2].astype(jnp.bfloat16),
                                 w1b[...], preferred_element_type=jnp.float32)

        def part_q(f):
            return part[f.dir.di, pl.ds(f.sub * Q, Q), :]

        def S(f, s):
            if s <= n_steps:
                src = sbuf.at[f.fi, s % 2]
            else:
                src = comm.at[f.fi, (s - 1) % 2]
            r = pltpu.make_async_remote_copy(
                src_ref=src, dst_ref=comm.at[f.fi, s % 2],
                send_sem=send_sems[f.fi].at[s % 2],
                recv_sem=recv_sems[f.fi].at[s % 2],
                device_id=(f.dst,), device_id_type=pl.DeviceIdType.MESH)
            r.start()
            f.rdma[s] = r

        def R(f, s):
            f.rdma[s].wait_recv()

        def WS(f, s):
            f.rdma[s].wait_send()

        def C(f):
            pl.semaphore_signal(cred_sems[f.fi], inc=1, device_id=(f.src,),
                                device_id_type=pl.DeviceIdType.MESH)

        def K(f):
            pl.semaphore_wait(cred_sems[f.fi], 1)

        def G(f, c, src):
            out_ref[pl.ds(c * M_CHUNK + f.qoff, Q), :] = jnp.dot(
                src[...].astype(jnp.bfloat16), w2b[...],
                preferred_element_type=jnp.float32)

        for d in dirs:
            start_load(d, 0, my)
        for d in dirs:
            gemm1(d, 0, d.rs_chunk(0))
            for f in flows:
                if f.dir is d:
                    sbuf[f.fi, 0] = part_q(f).astype(COMM_DTYPE)
                    S(f, 0)

        for s in range(n_steps):
            for d in dirs:
                nxt = d.rs_chunk(s + 1) if s + 1 < n_steps else None
                gemm1(d, s + 1, nxt)
            for f in flows:
                R(f, s)
                if s >= 1:
                    WS(f, s - 1)
                sbuf[f.fi, (s + 1) % 2] = (
                    comm[f.fi, s % 2].astype(jnp.float32) + part_q(f)
                ).astype(COMM_DTYPE)
                C(f)
                if s < n_steps - 1:
                    if s + 1 >= 2:
                        K(f)
                    S(f, s + 1)

        for f in flows:
            K(f)
            S(f, n_steps)
            WS(f, n_steps - 1)
            G(f, f.own, sbuf.at[f.fi, 1])

        for t in range(n_steps):
            for f in flows:
                s = n_steps + t
                R(f, s)
                WS(f, s)
                if 1 <= t <= n_steps - 2:
                    C(f)
                if t < n_steps - 1:
                    K(f)
                    S(f, s + 1)
                G(f, f.ag_chunk(t), comm.at[f.fi, s % 2])

    return pl.pallas_call(
        body,
        out_shape=jax.ShapeDtypeStruct((M, F), jnp.float32),
        in_specs=[
            pl.BlockSpec(memory_space=pl.ANY),
            pl.BlockSpec(memory_space=pltpu.VMEM),
            pl.BlockSpec(memory_space=pltpu.VMEM),
        ],
        out_specs=pl.BlockSpec(memory_space=pltpu.VMEM),
        scratch_shapes=[
            pltpu.VMEM((2, 2, 2 * Q, K), jnp.float32),
            pltpu.VMEM((2, 2 * Q, D), jnp.float32),
            pltpu.VMEM((N_FLOWS, 2, Q, D), COMM_DTYPE),
            pltpu.VMEM((N_FLOWS, 2, Q, D), COMM_DTYPE),
            pltpu.VMEM((K, D), jnp.bfloat16),
            pltpu.VMEM((D, F), jnp.bfloat16),
            *([pltpu.SemaphoreType.DMA((2,))] * 2),
            *([pltpu.SemaphoreType.DMA((2,))] * N_FLOWS),
            *([pltpu.SemaphoreType.DMA((2,))] * N_FLOWS),
            *([pltpu.SemaphoreType.REGULAR] * N_FLOWS),
        ],
        compiler_params=pltpu.CompilerParams(
            collective_id=0, vmem_limit_bytes=100 * 1024 * 1024),
    )(x, W1, W2)
